# baseline (speedup 1.0000x reference)
import functools

import jax
import jax.numpy as jnp
from jax import lax
from jax.experimental import pallas as pl
from jax.experimental.pallas import tpu as pltpu
from jax.experimental.pallas import tpu_sc as plsc

N = 10000
E = 320000
D = 128
R = 32
B = 4
BS = D // B

HD = D // 2
NS = 16
CH = 250
K = 80
RP = R + 1
NT = 1000
NPAD = 10240

_SC_PARAMS = pltpu.CompilerParams(use_tc_tiling_on_sc=False)


def _blockdiag(W, root):
    Wd = jnp.einsum('rbij,bc->rbicj', W, jnp.eye(B, dtype=W.dtype))
    return jnp.concatenate([Wd.reshape(R, D, D), root[None]], axis=0)


RB = 33


def _h_body(x_ref, w_ref, h_ref):
    for k in range(RB):
        h_ref[k] = jnp.dot(x_ref[...], w_ref[k],
                           preferred_element_type=jnp.float32)


def _transform(x, Wall):
    return pl.pallas_call(
        _h_body,
        grid=(N // NT, RP // RB),
        in_specs=[
            pl.BlockSpec((NT, D), lambda n, r: (n, 0)),
            pl.BlockSpec((RB, D, D), lambda n, r: (r, 0, 0)),
        ],
        out_specs=pl.BlockSpec((RB, NT, D), lambda n, r: (r, n, 0)),
        out_shape=jax.ShapeDtypeStruct((RP, N, D), jnp.float32),
    )(x, Wall)


def _deg_body(dsts, zdeg, deg_out, dst_v, ones_v, deg_sh):
    c = lax.axis_index("c")
    s = lax.axis_index("s")

    rows_per = NPAD // NS
    zsl = pl.ds(s * rows_per, rows_per)
    pltpu.sync_copy(zdeg.at[zsl], deg_sh.at[zsl])

    pltpu.sync_copy(dsts.at[s], dst_v)

    def ones_body(i, _):
        ones_v[i] = jnp.full((16,), 1.0, jnp.float32)
        return 0
    lax.fori_loop(0, K, ones_body, 0)

    plsc.subcore_barrier()

    half = CH // 2

    def chunk_body(j, _):
        pltpu.sync_copy(ones_v, deg_sh.at[dst_v.at[c * half + j]], add=True)
        return 0
    lax.fori_loop(0, half, chunk_body, 0)

    plsc.subcore_barrier()
    pltpu.sync_copy(deg_sh.at[zsl], deg_out.at[c, zsl])


def _sc_degrees(dsts):
    mesh = plsc.VectorSubcoreMesh(core_axis_name="c", subcore_axis_name="s")
    k = pl.kernel(
        _deg_body,
        out_type=jax.ShapeDtypeStruct((2, NPAD, 16), jnp.float32),
        mesh=mesh,
        scratch_types=[
            pltpu.VMEM((CH, K), jnp.int32),
            pltpu.VMEM((K, 16), jnp.float32),
            pltpu.VMEM_SHARED((NPAD, 16), jnp.float32),
        ],
        compiler_params=_SC_PARAMS,
    )
    return k(dsts, jnp.zeros((NPAD, 16), jnp.float32))


def _sc_body(table, srcs, rts, dsts, zrow, dep, agg_out,
             src_v, idx_v, dst_v, rows0, rows1, rows2, rows3, agg_sh, sem):
    del dep
    c = lax.axis_index("c")
    s = lax.axis_index("s")

    rows_per = NPAD // NS
    zsl = pl.ds(s * rows_per, rows_per)
    pltpu.sync_copy(zrow.at[zsl], agg_sh.at[zsl])

    pltpu.sync_copy(srcs.at[s], src_v)
    pltpu.sync_copy(rts.at[s], idx_v)
    pltpu.sync_copy(dsts.at[s], dst_v)

    def idx_body(j, _):
        for i in range(K // 16):
            sl = pl.ds(i * 16, 16)
            idx_v[j, sl] = (idx_v[j, sl] * N + src_v[j, sl]) * 2 + c
        return 0
    lax.fori_loop(0, CH, idx_body, 0)

    plsc.subcore_barrier()

    bufs = (rows0, rows1, rows2, rows3)
    pltpu.async_copy(table.at[idx_v.at[0]], rows0, sem)
    pltpu.async_copy(table.at[idx_v.at[1]], rows1, sem)
    pltpu.async_copy(table.at[idx_v.at[2]], rows2, sem)

    def quad_body(t, _):
        j = 4 * t
        for q in range(4):
            jj = j + q
            buf = bufs[q]
            nbuf = bufs[(q + 3) % 4]

            @pl.when(jj + 3 < CH - 2)
            def _(jj=jj, nbuf=nbuf):
                pltpu.async_copy(table.at[idx_v.at[jj + 3]], nbuf, sem)
            pltpu.make_async_copy(table.at[idx_v.at[jj]], buf, sem).wait()
            pltpu.sync_copy(buf, agg_sh.at[dst_v.at[jj]], add=True)
        return 0
    lax.fori_loop(0, CH // 4, quad_body, 0)

    pltpu.async_copy(table.at[idx_v.at[CH - 2]], rows0, sem)
    pltpu.async_copy(table.at[idx_v.at[CH - 1]], rows1, sem)
    pltpu.make_async_copy(table.at[idx_v.at[CH - 2]], rows0, sem).wait()
    pltpu.sync_copy(rows0, agg_sh.at[dst_v.at[CH - 2]], add=True)
    pltpu.make_async_copy(table.at[idx_v.at[CH - 1]], rows1, sem).wait()
    pltpu.sync_copy(rows1, agg_sh.at[dst_v.at[CH - 1]], add=True)

    plsc.subcore_barrier()
    pltpu.sync_copy(agg_sh.at[zsl], agg_out.at[c, zsl])


def _sc_gather_scatter(table, srcs, rts, dsts, zrow, dep):
    mesh = plsc.VectorSubcoreMesh(core_axis_name="c", subcore_axis_name="s")
    k = pl.kernel(
        _sc_body,
        out_type=jax.ShapeDtypeStruct((2, NPAD, HD), jnp.float32),
        mesh=mesh,
        scratch_types=[
            pltpu.VMEM((CH, K), jnp.int32),
            pltpu.VMEM((CH, K), jnp.int32),
            pltpu.VMEM((CH, K), jnp.int32),
            pltpu.VMEM((K, HD), jnp.float32),
            pltpu.VMEM((K, HD), jnp.float32),
            pltpu.VMEM((K, HD), jnp.float32),
            pltpu.VMEM((K, HD), jnp.float32),
            pltpu.VMEM_SHARED((NPAD, HD), jnp.float32),
            pltpu.SemaphoreType.DMA,
        ],
        compiler_params=_SC_PARAMS,
    )
    return k(table, srcs, rts, dsts, zrow, dep)


def _combine_body(relu, p_ref, dp_ref, rt_ref, b_ref, o_ref):
    agg = jnp.concatenate([p_ref[0], p_ref[1]], axis=-1)
    deg = dp_ref[0, :, 0:1] + dp_ref[1, :, 0:1]
    y = agg / jnp.maximum(deg, 1.0) + rt_ref[0] + b_ref[...]
    o_ref[...] = jnp.maximum(y, 0.0) if relu else y


def _combine(partials, degp, H, bias, relu):
    return pl.pallas_call(
        functools.partial(_combine_body, relu),
        grid=(N // NT,),
        in_specs=[
            pl.BlockSpec((2, NT, HD), lambda n: (0, n, 0)),
            pl.BlockSpec((2, NT, 16), lambda n: (0, n, 0)),
            pl.BlockSpec((1, NT, D), lambda n: (R, n, 0)),
            pl.BlockSpec((1, D), lambda n: (0, 0)),
        ],
        out_specs=pl.BlockSpec((NT, D), lambda n: (n, 0)),
        out_shape=jax.ShapeDtypeStruct((N, D), jnp.float32),
    )(partials, degp, H, bias.reshape(1, D))


def kernel(edge_index, edge_type, node_emb, W1, root1, b1, W2, root2, b2):
    srcs = edge_index[:, 0].reshape(NS, CH, K)
    dsts = edge_index[:, 1].reshape(NS, CH, K)
    rts = edge_type.reshape(NS, CH, K)

    Wall1 = _blockdiag(W1, root1)
    Wall2 = _blockdiag(W2, root2)

    degp = _sc_degrees(dsts)
    H1 = _transform(node_emb, Wall1)
    dep = degp[:1, :8, :16]
    zrow = jnp.zeros((NPAD, HD), jnp.float32)
    agg1 = _sc_gather_scatter(H1.reshape(RP * N * 2, HD), srcs, rts, dsts,
                              zrow, dep)
    x1 = _combine(agg1, degp, H1, b1, relu=True)

    H2 = _transform(x1, Wall2)
    agg2 = _sc_gather_scatter(H2.reshape(RP * N * 2, HD), srcs, rts, dsts,
                              zrow, dep)
    return _combine(agg2, degp, H2, b2, relu=False)


# speedup vs baseline: 1.0204x; 1.0204x over previous
import functools

import jax
import jax.numpy as jnp
from jax import lax
from jax.experimental import pallas as pl
from jax.experimental.pallas import tpu as pltpu
from jax.experimental.pallas import tpu_sc as plsc

N = 10000
E = 320000
D = 128
R = 32
B = 4
BS = D // B

HD = D // 2
NS = 16
CH = 250
K = 80
RP = R + 1
NT = 2000
NPAD = 10240

_SC_PARAMS = pltpu.CompilerParams(use_tc_tiling_on_sc=False)


def _blockdiag(W, root):
    Wd = jnp.einsum('rbij,bc->rbicj', W, jnp.eye(B, dtype=W.dtype))
    return jnp.concatenate([Wd.reshape(R, D, D), root[None]], axis=0)


RB = 11


def _h_body(x_ref, w_ref, h_ref):
    for k in range(RB):
        h_ref[k] = jnp.dot(x_ref[...], w_ref[k],
                           preferred_element_type=jnp.float32)


def _transform(x, Wall):
    return pl.pallas_call(
        _h_body,
        grid=(N // NT, RP // RB),
        in_specs=[
            pl.BlockSpec((NT, D), lambda n, r: (n, 0)),
            pl.BlockSpec((RB, D, D), lambda n, r: (r, 0, 0)),
        ],
        out_specs=pl.BlockSpec((RB, NT, D), lambda n, r: (r, n, 0)),
        out_shape=jax.ShapeDtypeStruct((RP, N, D), jnp.float32),
    )(x, Wall)


def _deg_body(dsts, zdeg, deg_out, dst_v, ones_v, deg_sh):
    c = lax.axis_index("c")
    s = lax.axis_index("s")

    rows_per = NPAD // NS
    zsl = pl.ds(s * rows_per, rows_per)
    pltpu.sync_copy(zdeg.at[zsl], deg_sh.at[zsl])

    pltpu.sync_copy(dsts.at[s], dst_v)

    def ones_body(i, _):
        ones_v[i] = jnp.full((16,), 1.0, jnp.float32)
        return 0
    lax.fori_loop(0, K, ones_body, 0)

    plsc.subcore_barrier()

    half = CH // 2

    def chunk_body(j, _):
        pltpu.sync_copy(ones_v, deg_sh.at[dst_v.at[c * half + j]], add=True)
        return 0
    lax.fori_loop(0, half, chunk_body, 0)

    plsc.subcore_barrier()
    pltpu.sync_copy(deg_sh.at[zsl], deg_out.at[c, zsl])


def _sc_degrees(dsts):
    mesh = plsc.VectorSubcoreMesh(core_axis_name="c", subcore_axis_name="s")
    k = pl.kernel(
        _deg_body,
        out_type=jax.ShapeDtypeStruct((2, NPAD, 16), jnp.float32),
        mesh=mesh,
        scratch_types=[
            pltpu.VMEM((CH, K), jnp.int32),
            pltpu.VMEM((K, 16), jnp.float32),
            pltpu.VMEM_SHARED((NPAD, 16), jnp.float32),
        ],
        compiler_params=_SC_PARAMS,
    )
    return k(dsts, jnp.zeros((NPAD, 16), jnp.float32))


def _sc_body(table, srcs, rts, dsts, zrow, dep, agg_out,
             src_v, idx_v, dst_v, rows0, rows1, rows2, rows3, agg_sh, sem):
    del dep
    c = lax.axis_index("c")
    s = lax.axis_index("s")

    rows_per = NPAD // NS
    zsl = pl.ds(s * rows_per, rows_per)
    pltpu.sync_copy(zrow.at[zsl], agg_sh.at[zsl])

    pltpu.sync_copy(srcs.at[s], src_v)
    pltpu.sync_copy(rts.at[s], idx_v)
    pltpu.sync_copy(dsts.at[s], dst_v)

    def idx_body(j, _):
        for i in range(K // 16):
            sl = pl.ds(i * 16, 16)
            idx_v[j, sl] = (idx_v[j, sl] * N + src_v[j, sl]) * 2 + c
        return 0
    lax.fori_loop(0, CH, idx_body, 0)

    plsc.subcore_barrier()

    bufs = (rows0, rows1, rows2, rows3)
    pltpu.async_copy(table.at[idx_v.at[0]], rows0, sem)
    pltpu.async_copy(table.at[idx_v.at[1]], rows1, sem)
    pltpu.async_copy(table.at[idx_v.at[2]], rows2, sem)

    def quad_body(t, _):
        j = 4 * t
        for q in range(4):
            jj = j + q
            buf = bufs[q]
            nbuf = bufs[(q + 3) % 4]

            @pl.when(jj + 3 < CH - 2)
            def _(jj=jj, nbuf=nbuf):
                pltpu.async_copy(table.at[idx_v.at[jj + 3]], nbuf, sem)
            pltpu.make_async_copy(table.at[idx_v.at[jj]], buf, sem).wait()
            pltpu.sync_copy(buf, agg_sh.at[dst_v.at[jj]], add=True)
        return 0
    lax.fori_loop(0, CH // 4, quad_body, 0)

    pltpu.async_copy(table.at[idx_v.at[CH - 2]], rows0, sem)
    pltpu.async_copy(table.at[idx_v.at[CH - 1]], rows1, sem)
    pltpu.make_async_copy(table.at[idx_v.at[CH - 2]], rows0, sem).wait()
    pltpu.sync_copy(rows0, agg_sh.at[dst_v.at[CH - 2]], add=True)
    pltpu.make_async_copy(table.at[idx_v.at[CH - 1]], rows1, sem).wait()
    pltpu.sync_copy(rows1, agg_sh.at[dst_v.at[CH - 1]], add=True)

    plsc.subcore_barrier()
    pltpu.sync_copy(agg_sh.at[zsl], agg_out.at[c, zsl])


def _sc_gather_scatter(table, srcs, rts, dsts, zrow, dep):
    mesh = plsc.VectorSubcoreMesh(core_axis_name="c", subcore_axis_name="s")
    k = pl.kernel(
        _sc_body,
        out_type=jax.ShapeDtypeStruct((2, NPAD, HD), jnp.float32),
        mesh=mesh,
        scratch_types=[
            pltpu.VMEM((CH, K), jnp.int32),
            pltpu.VMEM((CH, K), jnp.int32),
            pltpu.VMEM((CH, K), jnp.int32),
            pltpu.VMEM((K, HD), jnp.float32),
            pltpu.VMEM((K, HD), jnp.float32),
            pltpu.VMEM((K, HD), jnp.float32),
            pltpu.VMEM((K, HD), jnp.float32),
            pltpu.VMEM_SHARED((NPAD, HD), jnp.float32),
            pltpu.SemaphoreType.DMA,
        ],
        compiler_params=_SC_PARAMS,
    )
    return k(table, srcs, rts, dsts, zrow, dep)


def _combine_body(relu, p_ref, dp_ref, rt_ref, b_ref, o_ref):
    agg = jnp.concatenate([p_ref[0], p_ref[1]], axis=-1)
    deg = dp_ref[0, :, 0:1] + dp_ref[1, :, 0:1]
    y = agg / jnp.maximum(deg, 1.0) + rt_ref[0] + b_ref[...]
    o_ref[...] = jnp.maximum(y, 0.0) if relu else y


def _combine(partials, degp, H, bias, relu):
    return pl.pallas_call(
        functools.partial(_combine_body, relu),
        grid=(N // NT,),
        in_specs=[
            pl.BlockSpec((2, NT, HD), lambda n: (0, n, 0)),
            pl.BlockSpec((2, NT, 16), lambda n: (0, n, 0)),
            pl.BlockSpec((1, NT, D), lambda n: (R, n, 0)),
            pl.BlockSpec((1, D), lambda n: (0, 0)),
        ],
        out_specs=pl.BlockSpec((NT, D), lambda n: (n, 0)),
        out_shape=jax.ShapeDtypeStruct((N, D), jnp.float32),
    )(partials, degp, H, bias.reshape(1, D))


def kernel(edge_index, edge_type, node_emb, W1, root1, b1, W2, root2, b2):
    srcs = edge_index[:, 0].reshape(NS, CH, K)
    dsts = edge_index[:, 1].reshape(NS, CH, K)
    rts = edge_type.reshape(NS, CH, K)

    Wall1 = _blockdiag(W1, root1)
    Wall2 = _blockdiag(W2, root2)

    degp = _sc_degrees(dsts)
    H1 = _transform(node_emb, Wall1)
    dep = degp[:1, :8, :16]
    zrow = jnp.zeros((NPAD, HD), jnp.float32)
    agg1 = _sc_gather_scatter(H1.reshape(RP * N * 2, HD), srcs, rts, dsts,
                              zrow, dep)
    x1 = _combine(agg1, degp, H1, b1, relu=True)

    H2 = _transform(x1, Wall2)
    agg2 = _sc_gather_scatter(H2.reshape(RP * N * 2, HD), srcs, rts, dsts,
                              zrow, dep)
    return _combine(agg2, degp, H2, b2, relu=False)


# speedup vs baseline: 1.0767x; 1.0552x over previous
import functools

import jax
import jax.numpy as jnp
from jax import lax
from jax.experimental import pallas as pl
from jax.experimental.pallas import tpu as pltpu
from jax.experimental.pallas import tpu_sc as plsc

N = 10000
E = 320000
D = 128
R = 32
B = 4
BS = D // B

HD = D // 2
NS = 16
CH = 250
K = 80
RP = R + 1
NT = 2000
NPAD = 10240

_SC_PARAMS = pltpu.CompilerParams(use_tc_tiling_on_sc=False)


RB = 8


def _h_body(x_ref, w_ref, h_ref):
    z = jnp.zeros((BS, BS), jnp.float32)
    for k in range(RB):
        rows = [
            jnp.concatenate(
                [w_ref[k, bb] if cc == bb else z for cc in range(B)], axis=1)
            for bb in range(B)
        ]
        bd = jnp.concatenate(rows, axis=0)
        h_ref[k] = jnp.dot(x_ref[...], bd, preferred_element_type=jnp.float32)


def _transform(x, W):
    return pl.pallas_call(
        _h_body,
        grid=(N // NT, R // RB),
        in_specs=[
            pl.BlockSpec((NT, D), lambda n, r: (n, 0)),
            pl.BlockSpec((RB, B, BS, BS), lambda n, r: (r, 0, 0, 0)),
        ],
        out_specs=pl.BlockSpec((RB, NT, D), lambda n, r: (r, n, 0)),
        out_shape=jax.ShapeDtypeStruct((R, N, D), jnp.float32),
    )(x, W)


def _deg_body(dsts, zdeg, deg_out, dst_v, ones_v, deg_sh):
    c = lax.axis_index("c")
    s = lax.axis_index("s")

    rows_per = NPAD // NS
    zsl = pl.ds(s * rows_per, rows_per)
    pltpu.sync_copy(zdeg.at[zsl], deg_sh.at[zsl])

    pltpu.sync_copy(dsts.at[s], dst_v)

    def ones_body(i, _):
        ones_v[i] = jnp.full((16,), 1.0, jnp.float32)
        return 0
    lax.fori_loop(0, K, ones_body, 0)

    plsc.subcore_barrier()

    half = CH // 2

    def chunk_body(j, _):
        pltpu.sync_copy(ones_v, deg_sh.at[dst_v.at[c * half + j]], add=True)
        return 0
    lax.fori_loop(0, half, chunk_body, 0)

    plsc.subcore_barrier()
    pltpu.sync_copy(deg_sh.at[zsl], deg_out.at[c, zsl])


def _sc_degrees(dsts):
    mesh = plsc.VectorSubcoreMesh(core_axis_name="c", subcore_axis_name="s")
    k = pl.kernel(
        _deg_body,
        out_type=jax.ShapeDtypeStruct((2, NPAD, 16), jnp.float32),
        mesh=mesh,
        scratch_types=[
            pltpu.VMEM((CH, K), jnp.int32),
            pltpu.VMEM((K, 16), jnp.float32),
            pltpu.VMEM_SHARED((NPAD, 16), jnp.float32),
        ],
        compiler_params=_SC_PARAMS,
    )
    return k(dsts, jnp.zeros((NPAD, 16), jnp.float32))


def _sc_body(table, srcs, rts, dsts, zrow, dep, agg_out,
             src_v, idx_v, dst_v, rows0, rows1, rows2, rows3, agg_sh, sem):
    del dep
    c = lax.axis_index("c")
    s = lax.axis_index("s")

    rows_per = NPAD // NS
    zsl = pl.ds(s * rows_per, rows_per)
    pltpu.sync_copy(zrow.at[zsl], agg_sh.at[zsl])
    pltpu.sync_copy(srcs.at[s], src_v)
    pltpu.sync_copy(rts.at[s], idx_v)
    pltpu.sync_copy(dsts.at[s], dst_v)

    def idx_body(j, _):
        for i in range(K // 16):
            sl = pl.ds(i * 16, 16)
            idx_v[j, sl] = (idx_v[j, sl] * N + src_v[j, sl]) * 2 + c
        return 0
    lax.fori_loop(0, CH, idx_body, 0)

    plsc.subcore_barrier()

    bufs = (rows0, rows1, rows2, rows3)
    pltpu.async_copy(table.at[idx_v.at[0]], rows0, sem)
    pltpu.async_copy(table.at[idx_v.at[1]], rows1, sem)
    pltpu.async_copy(table.at[idx_v.at[2]], rows2, sem)

    def quad_body(t, _):
        j = 4 * t
        for q in range(4):
            jj = j + q
            buf = bufs[q]
            nbuf = bufs[(q + 3) % 4]

            @pl.when(jj + 3 < CH - 2)
            def _(jj=jj, nbuf=nbuf):
                pltpu.async_copy(table.at[idx_v.at[jj + 3]], nbuf, sem)
            pltpu.make_async_copy(table.at[idx_v.at[jj]], buf, sem).wait()
            pltpu.sync_copy(buf, agg_sh.at[dst_v.at[jj]], add=True)
        return 0
    lax.fori_loop(0, CH // 4, quad_body, 0)

    pltpu.async_copy(table.at[idx_v.at[CH - 2]], rows0, sem)
    pltpu.async_copy(table.at[idx_v.at[CH - 1]], rows1, sem)
    pltpu.make_async_copy(table.at[idx_v.at[CH - 2]], rows0, sem).wait()
    pltpu.sync_copy(rows0, agg_sh.at[dst_v.at[CH - 2]], add=True)
    pltpu.make_async_copy(table.at[idx_v.at[CH - 1]], rows1, sem).wait()
    pltpu.sync_copy(rows1, agg_sh.at[dst_v.at[CH - 1]], add=True)

    plsc.subcore_barrier()
    pltpu.sync_copy(agg_sh.at[zsl], agg_out.at[c, zsl])


def _sc_gather_scatter(table, srcs, rts, dsts, zrow, dep):
    mesh = plsc.VectorSubcoreMesh(core_axis_name="c", subcore_axis_name="s")
    k = pl.kernel(
        _sc_body,
        out_type=jax.ShapeDtypeStruct((2, NPAD, HD), jnp.float32),
        mesh=mesh,
        scratch_types=[
            pltpu.VMEM((CH, K), jnp.int32),
            pltpu.VMEM((CH, K), jnp.int32),
            pltpu.VMEM((CH, K), jnp.int32),
            pltpu.VMEM((K, HD), jnp.float32),
            pltpu.VMEM((K, HD), jnp.float32),
            pltpu.VMEM((K, HD), jnp.float32),
            pltpu.VMEM((K, HD), jnp.float32),
            pltpu.VMEM_SHARED((NPAD, HD), jnp.float32),
            pltpu.SemaphoreType.DMA,
        ],
        compiler_params=_SC_PARAMS,
    )
    return k(table, srcs, rts, dsts, zrow, dep)


def _combine_body(relu, p_ref, dp_ref, x_ref, root_ref, b_ref, o_ref):
    agg = jnp.concatenate([p_ref[0], p_ref[1]], axis=-1)
    deg = dp_ref[0, :, 0:1] + dp_ref[1, :, 0:1]
    rootterm = jnp.dot(x_ref[...], root_ref[...],
                       preferred_element_type=jnp.float32)
    y = agg / jnp.maximum(deg, 1.0) + rootterm + b_ref[...]
    o_ref[...] = jnp.maximum(y, 0.0) if relu else y


def _combine(partials, degp, x, root, bias, relu):
    return pl.pallas_call(
        functools.partial(_combine_body, relu),
        grid=(N // NT,),
        in_specs=[
            pl.BlockSpec((2, NT, HD), lambda n: (0, n, 0)),
            pl.BlockSpec((2, NT, 16), lambda n: (0, n, 0)),
            pl.BlockSpec((NT, D), lambda n: (n, 0)),
            pl.BlockSpec((D, D), lambda n: (0, 0)),
            pl.BlockSpec((1, D), lambda n: (0, 0)),
        ],
        out_specs=pl.BlockSpec((NT, D), lambda n: (n, 0)),
        out_shape=jax.ShapeDtypeStruct((N, D), jnp.float32),
    )(partials, degp, x, root, bias.reshape(1, D))


def kernel(edge_index, edge_type, node_emb, W1, root1, b1, W2, root2, b2):
    srcs = edge_index[:, 0].reshape(NS, CH, K)
    dsts = edge_index[:, 1].reshape(NS, CH, K)
    rts = edge_type.reshape(NS, CH, K)

    degp = _sc_degrees(dsts)
    H1 = _transform(node_emb, W1)
    dep = degp[:1, :8, :16]
    zrow = jnp.zeros((NPAD, HD), jnp.float32)
    agg1 = _sc_gather_scatter(H1.reshape(R * N * 2, HD), srcs, rts, dsts,
                              zrow, dep)
    x1 = _combine(agg1, degp, node_emb, root1, b1, relu=True)

    H2 = _transform(x1, W2)
    agg2 = _sc_gather_scatter(H2.reshape(R * N * 2, HD), srcs, rts, dsts,
                              zrow, dep)
    return _combine(agg2, degp, x1, root2, b2, relu=False)


# speedup vs baseline: 1.1140x; 1.0346x over previous
import functools

import jax
import jax.numpy as jnp
from jax import lax
from jax.experimental import pallas as pl
from jax.experimental.pallas import tpu as pltpu
from jax.experimental.pallas import tpu_sc as plsc

N = 10000
E = 320000
D = 128
R = 32
B = 4
BS = D // B

HD = D // 2
NS = 16
CH = 250
K = 80
RP = R + 1
NT = 2000
NPAD = 10240

_SC_PARAMS = pltpu.CompilerParams(use_tc_tiling_on_sc=False)


RB = 8


def _h_body(x_ref, w_ref, h_ref):
    z = jnp.zeros((BS, BS), jnp.float32)
    for k in range(RB):
        rows = [
            jnp.concatenate(
                [w_ref[k, bb] if cc == bb else z for cc in range(B)], axis=1)
            for bb in range(B)
        ]
        bd = jnp.concatenate(rows, axis=0)
        h_ref[k] = jnp.dot(x_ref[...], bd, preferred_element_type=jnp.float32)


def _transform(x, W):
    return pl.pallas_call(
        _h_body,
        grid=(N // NT, R // RB),
        in_specs=[
            pl.BlockSpec((NT, D), lambda n, r: (n, 0)),
            pl.BlockSpec((RB, B, BS, BS), lambda n, r: (r, 0, 0, 0)),
        ],
        out_specs=pl.BlockSpec((RB, NT, D), lambda n, r: (r, n, 0)),
        out_shape=jax.ShapeDtypeStruct((R, N, D), jnp.float32),
    )(x, W)


def _deg_body(dsts, zdeg, deg_out, dst_v, ones_v, deg_sh):
    c = lax.axis_index("c")
    s = lax.axis_index("s")

    rows_per = NPAD // NS
    zsl = pl.ds(s * rows_per, rows_per)
    pltpu.sync_copy(zdeg.at[zsl], deg_sh.at[zsl])

    pltpu.sync_copy(dsts.at[s], dst_v)

    def ones_body(i, _):
        ones_v[i] = jnp.full((16,), 1.0, jnp.float32)
        return 0
    lax.fori_loop(0, K, ones_body, 0)

    plsc.subcore_barrier()

    half = CH // 2

    def chunk_body(j, _):
        pltpu.sync_copy(ones_v, deg_sh.at[dst_v.at[c * half + j]], add=True)
        return 0
    lax.fori_loop(0, half, chunk_body, 0)

    plsc.subcore_barrier()
    pltpu.sync_copy(deg_sh.at[zsl], deg_out.at[c, zsl])


def _sc_degrees(dsts):
    mesh = plsc.VectorSubcoreMesh(core_axis_name="c", subcore_axis_name="s")
    k = pl.kernel(
        _deg_body,
        out_type=jax.ShapeDtypeStruct((2, NPAD, 16), jnp.float32),
        mesh=mesh,
        scratch_types=[
            pltpu.VMEM((CH, K), jnp.int32),
            pltpu.VMEM((K, 16), jnp.float32),
            pltpu.VMEM_SHARED((NPAD, 16), jnp.float32),
        ],
        compiler_params=_SC_PARAMS,
    )
    return k(dsts, jnp.zeros((NPAD, 16), jnp.float32))


def _sc_body(table, srcs, rts, dsts, zrow, dep, agg_out,
             src_v, idx_v, dst_v, rows0, rows1, rows2, rows3, agg_sh, sem):
    del dep
    c = lax.axis_index("c")
    s = lax.axis_index("s")

    rows_per = NPAD // NS
    zsl = pl.ds(s * rows_per, rows_per)
    pltpu.sync_copy(zrow.at[zsl], agg_sh.at[zsl])
    pltpu.sync_copy(srcs.at[s], src_v)
    pltpu.sync_copy(rts.at[s], idx_v)
    pltpu.sync_copy(dsts.at[s], dst_v)

    def idx_body(j, _):
        for i in range(K // 16):
            sl = pl.ds(i * 16, 16)
            idx_v[j, sl] = (idx_v[j, sl] * N + src_v[j, sl]) * 2 + c
        return 0
    lax.fori_loop(0, CH, idx_body, 0)

    plsc.subcore_barrier()

    bufs = (rows0, rows1, rows2, rows3)
    pltpu.async_copy(table.at[idx_v.at[0]], rows0, sem)
    pltpu.async_copy(table.at[idx_v.at[1]], rows1, sem)
    pltpu.async_copy(table.at[idx_v.at[2]], rows2, sem)

    def quad_body(t, _):
        j = 4 * t
        for q in range(4):
            jj = j + q
            buf = bufs[q]
            nbuf = bufs[(q + 3) % 4]

            @pl.when(jj + 3 < CH - 2)
            def _(jj=jj, nbuf=nbuf):
                pltpu.async_copy(table.at[idx_v.at[jj + 3]], nbuf, sem)
            pltpu.make_async_copy(table.at[idx_v.at[jj]], buf, sem).wait()
            pltpu.sync_copy(buf, agg_sh.at[dst_v.at[jj]], add=True)
        return 0
    lax.fori_loop(0, CH // 4, quad_body, 0)

    pltpu.async_copy(table.at[idx_v.at[CH - 2]], rows0, sem)
    pltpu.async_copy(table.at[idx_v.at[CH - 1]], rows1, sem)
    pltpu.make_async_copy(table.at[idx_v.at[CH - 2]], rows0, sem).wait()
    pltpu.sync_copy(rows0, agg_sh.at[dst_v.at[CH - 2]], add=True)
    pltpu.make_async_copy(table.at[idx_v.at[CH - 1]], rows1, sem).wait()
    pltpu.sync_copy(rows1, agg_sh.at[dst_v.at[CH - 1]], add=True)

    plsc.subcore_barrier()
    pltpu.sync_copy(agg_sh.at[zsl], agg_out.at[c, zsl])


def _sc_gather_scatter(table, srcs, rts, dsts, zrow, dep):
    mesh = plsc.VectorSubcoreMesh(core_axis_name="c", subcore_axis_name="s")
    k = pl.kernel(
        _sc_body,
        out_type=jax.ShapeDtypeStruct((2, NPAD, HD), jnp.float32),
        mesh=mesh,
        scratch_types=[
            pltpu.VMEM((CH, K), jnp.int32),
            pltpu.VMEM((CH, K), jnp.int32),
            pltpu.VMEM((CH, K), jnp.int32),
            pltpu.VMEM((K, HD), jnp.float32),
            pltpu.VMEM((K, HD), jnp.float32),
            pltpu.VMEM((K, HD), jnp.float32),
            pltpu.VMEM((K, HD), jnp.float32),
            pltpu.VMEM_SHARED((NPAD, HD), jnp.float32),
            pltpu.SemaphoreType.DMA,
        ],
        compiler_params=_SC_PARAMS,
    )
    return k(table, srcs, rts, dsts, zrow, dep)


def _combine_body(relu, p_ref, dp_ref, x_ref, root_ref, b_ref, o_ref):
    a0 = p_ref[0]
    a1 = p_ref[1]
    even = jnp.concatenate([a0[:, :HD], a1[:, :HD]], axis=1)
    odd = jnp.concatenate([a0[:, HD:], a1[:, HD:]], axis=1)
    agg = jnp.stack([even, odd], axis=1).reshape(NT, D)
    deg = dp_ref[0, :, 0:1] + dp_ref[1, :, 0:1]
    rootterm = jnp.dot(x_ref[...], root_ref[...],
                       preferred_element_type=jnp.float32)
    y = agg / jnp.maximum(deg, 1.0) + rootterm + b_ref[...]
    o_ref[...] = jnp.maximum(y, 0.0) if relu else y


def _combine(partials, degp, x, root, bias, relu):
    return pl.pallas_call(
        functools.partial(_combine_body, relu),
        grid=(N // NT,),
        in_specs=[
            pl.BlockSpec((2, NT // 2, D), lambda n: (0, n, 0)),
            pl.BlockSpec((2, NT, 16), lambda n: (0, n, 0)),
            pl.BlockSpec((NT, D), lambda n: (n, 0)),
            pl.BlockSpec((D, D), lambda n: (0, 0)),
            pl.BlockSpec((1, D), lambda n: (0, 0)),
        ],
        out_specs=pl.BlockSpec((NT, D), lambda n: (n, 0)),
        out_shape=jax.ShapeDtypeStruct((N, D), jnp.float32),
    )(partials.reshape(2, NPAD // 2, D), degp, x, root,
      bias.reshape(1, D))


def kernel(edge_index, edge_type, node_emb, W1, root1, b1, W2, root2, b2):
    srcs = edge_index[:, 0].reshape(NS, CH, K)
    dsts = edge_index[:, 1].reshape(NS, CH, K)
    rts = edge_type.reshape(NS, CH, K)

    degp = _sc_degrees(dsts)
    H1 = _transform(node_emb, W1)
    dep = degp[:1, :8, :16]
    zrow = jnp.zeros((NPAD, HD), jnp.float32)
    agg1 = _sc_gather_scatter(H1.reshape(R * N * 2, HD), srcs, rts, dsts,
                              zrow, dep)
    x1 = _combine(agg1, degp, node_emb, root1, b1, relu=True)

    H2 = _transform(x1, W2)
    agg2 = _sc_gather_scatter(H2.reshape(R * N * 2, HD), srcs, rts, dsts,
                              zrow, dep)
    return _combine(agg2, degp, x1, root2, b2, relu=False)
